# baseline (speedup 1.0000x reference)
import functools

import jax
import jax.numpy as jnp
from jax import lax
from jax.experimental import pallas as pl
from jax.experimental.pallas import tpu as pltpu
from jax.experimental.pallas import tpu_sc as plsc

EMBED = 64
SCALE = 8.0
LANES = 16
BBLK = 128


def kernel(x, table):
    B, L = x.shape
    V = table.shape[0]
    t2 = table.reshape(V // 2, 2 * EMBED)
    xt = x.T

    mesh = plsc.VectorSubcoreMesh(core_axis_name="c", subcore_axis_name="s")

    @functools.partial(
        pl.kernel,
        out_type=jax.ShapeDtypeStruct((L, EMBED, B), jnp.float32),
        mesh=mesh,
        compiler_params=pltpu.CompilerParams(
            use_tc_tiling_on_sc=True, needs_layout_passes=False
        ),
        scratch_types=[
            pltpu.VMEM((BBLK,), jnp.int32),
            pltpu.VMEM((BBLK,), jnp.int32),
            pltpu.VMEM((BBLK, 2 * EMBED), jnp.float32),
        ],
    )
    def gather_scale(xt_hbm, t_hbm, o_hbm, jbuf, cbuf, gbuf):
        def body(i_vmem, o_vmem):
            riota = lax.iota(jnp.int32, LANES)

            @pl.loop(0, BBLK, step=LANES)
            def _prep(b0):
                idx = i_vmem.at[0, pl.ds(b0, LANES)][...]
                jbuf.at[pl.ds(b0, LANES)][...] = lax.shift_right_logical(idx, 1)
                cbuf.at[pl.ds(b0, LANES)][...] = (idx & 1) * EMBED

            pltpu.sync_copy(t_hbm.at[jbuf], gbuf)

            for k in range(BBLK // LANES):
                rows = riota + (k * LANES)
                cbase = cbuf.at[pl.ds(k * LANES, LANES)][...]

                @pl.loop(0, EMBED)
                def _emit(e):
                    vals = plsc.load_gather(gbuf, [rows, cbase + e])
                    o_vmem.at[0, e, pl.ds(k * LANES, LANES)][...] = vals * SCALE

        pltpu.emit_pipeline(
            body,
            grid=(L, B // BBLK),
            in_specs=[
                pl.BlockSpec((1, BBLK), index_map=lambda l, bt: (l, bt)),
            ],
            out_specs=[
                pl.BlockSpec((1, EMBED, BBLK), index_map=lambda l, bt: (l, 0, bt)),
            ],
            core_axis_name=("c", "s"),
            dimension_semantics=(pltpu.PARALLEL, pltpu.PARALLEL),
        )(xt_hbm, o_hbm)

    out = gather_scale(xt, t2)
    return out.transpose(2, 0, 1)


# speedup vs baseline: 1.4465x; 1.4465x over previous
import functools

import jax
import jax.numpy as jnp
from jax import lax
from jax.experimental import pallas as pl
from jax.experimental.pallas import tpu as pltpu
from jax.experimental.pallas import tpu_sc as plsc

EMBED = 64
SCALE = 8.0
LANES = 16
CHUNK = 128
NWORK = 32

_DNUMS = lax.GatherDimensionNumbers(
    offset_dims=(), collapsed_slice_dims=(0,), start_index_map=(0,)
)


def kernel(x, table):
    B, L = x.shape
    N = B * L
    V = table.shape[0]
    per_w = N // NWORK
    n_chunks = per_w // CHUNK

    t2 = table.reshape(V // 2, 2 * EMBED)
    idx = x.reshape(N // CHUNK, CHUNK)
    rows_w = per_w // CHUNK

    mesh = plsc.VectorSubcoreMesh(core_axis_name="c", subcore_axis_name="s")

    @functools.partial(
        pl.kernel,
        out_type=jax.ShapeDtypeStruct((N, EMBED), jnp.float32),
        mesh=mesh,
        compiler_params=pltpu.CompilerParams(
            use_tc_tiling_on_sc=True, needs_layout_passes=False
        ),
        scratch_types=[
            pltpu.VMEM((rows_w, CHUNK), jnp.int32),
            pltpu.VMEM((2, CHUNK), jnp.int32),
            pltpu.VMEM((2, CHUNK), jnp.int32),
            pltpu.VMEM((2, CHUNK, 2 * EMBED), jnp.float32),
            pltpu.VMEM((2, CHUNK, EMBED), jnp.float32),
            pltpu.SemaphoreType.DMA,
            pltpu.SemaphoreType.DMA,
            pltpu.SemaphoreType.DMA,
            pltpu.SemaphoreType.DMA,
            pltpu.SemaphoreType.DMA,
        ],
    )
    def gather_scale(i_hbm, t_hbm, o_hbm, iv, jb, cb, gb, ob, sem_i, sg0, sg1,
                     so0, so1):
        wid = lax.axis_index("s") * 2 + lax.axis_index("c")
        base = wid * per_w
        riota = lax.iota(jnp.int32, LANES)
        sem_g = (sg0, sg1)
        sem_o = (so0, so1)

        pltpu.async_copy(
            i_hbm.at[pl.ds(wid * rows_w, rows_w), :], iv, sem_i
        ).wait()

        def prep(chunk, slot):
            @pl.loop(0, CHUNK, step=LANES)
            def _p(m):
                ival = iv.at[chunk, pl.ds(m, LANES)][...]
                jb.at[slot, pl.ds(m, LANES)][...] = lax.shift_right_logical(
                    ival, 1)
                cb.at[slot, pl.ds(m, LANES)][...] = (ival & 1) * EMBED

        def start_gather(slot):
            pltpu.async_copy(t_hbm.at[jb.at[slot]], gb.at[slot], sem_g[slot])

        def wait_gather(slot):
            pltpu.make_async_copy(
                t_hbm.at[jb.at[slot]], gb.at[slot], sem_g[slot]).wait()

        def extract(slot):
            @pl.loop(0, CHUNK)
            def _r(r):
                cb16 = cb.at[slot, pl.ds((r // LANES) * LANES, LANES)][...]
                sel = jnp.broadcast_to(r % LANES, (LANES,)).astype(jnp.int32)
                cbase = lax.gather(
                    cb16, sel[:, None], _DNUMS, (1,),
                    mode=lax.GatherScatterMode.PROMISE_IN_BOUNDS,
                )
                rvec = jnp.broadcast_to(r, (LANES,)).astype(jnp.int32)
                for k in range(EMBED // LANES):
                    col = cbase + (riota + k * LANES)
                    vals = plsc.load_gather(gb.at[slot], [rvec, col])
                    ob.at[slot, r, pl.ds(k * LANES, LANES)][...] = vals * SCALE

        def start_out(chunk, slot):
            pltpu.async_copy(
                ob.at[slot],
                o_hbm.at[pl.ds(base + chunk * CHUNK, CHUNK), :],
                sem_o[slot],
            )

        def wait_out(chunk, slot):
            pltpu.make_async_copy(
                ob.at[slot],
                o_hbm.at[pl.ds(base + chunk * CHUNK, CHUNK), :],
                sem_o[slot],
            ).wait()

        prep(0, 0)
        start_gather(0)

        @pl.loop(0, n_chunks // 2)
        def _t(t):
            for b in range(2):
                i = t * 2 + b
                nxt = 1 - b

                @pl.when(i + 1 < n_chunks)
                def _():
                    prep(i + 1, nxt)
                    start_gather(nxt)

                wait_gather(b)

                @pl.when(i >= 2)
                def _():
                    wait_out(i - 2, b)

                extract(b)
                start_out(i, b)

        for b in range(2):
            wait_out(n_chunks - 2 + b, b)

    out = gather_scale(idx, t2)
    return out.reshape(B, L, EMBED)


# speedup vs baseline: 2.3520x; 1.6259x over previous
import functools

import jax
import jax.numpy as jnp
from jax import lax
from jax.experimental import pallas as pl
from jax.experimental.pallas import tpu as pltpu
from jax.experimental.pallas import tpu_sc as plsc

EMBED = 64
SCALE = 8.0
LANES = 16
CHUNK = 128
NWORK = 32


def kernel(x, table):
    B, L = x.shape
    N = B * L
    per_w = N // NWORK
    n_chunks = per_w // CHUNK
    rows_w = per_w // CHUNK

    idx = x.reshape(N // CHUNK, CHUNK)

    mesh = plsc.VectorSubcoreMesh(core_axis_name="c", subcore_axis_name="s")

    @functools.partial(
        pl.kernel,
        out_type=jax.ShapeDtypeStruct((N, 2 * EMBED), jnp.float32),
        mesh=mesh,
        compiler_params=pltpu.CompilerParams(use_tc_tiling_on_sc=False),
        scratch_types=[
            pltpu.VMEM((rows_w, CHUNK), jnp.int32),
            pltpu.VMEM((2, CHUNK, EMBED), jnp.float32),
            pltpu.VMEM((2, CHUNK, EMBED), jnp.float32),
            pltpu.SemaphoreType.DMA,
            pltpu.SemaphoreType.DMA,
            pltpu.SemaphoreType.DMA,
            pltpu.SemaphoreType.DMA,
            pltpu.SemaphoreType.DMA,
        ],
    )
    def gather_scale(i_hbm, t_hbm, o_hbm, iv, gb, ob, sem_i, sg0, sg1, so0,
                     so1):
        wid = lax.axis_index("s") * 2 + lax.axis_index("c")
        base = wid * per_w
        sem_g = (sg0, sg1)
        sem_o = (so0, so1)

        pltpu.async_copy(
            i_hbm.at[pl.ds(wid * rows_w, rows_w), :], iv, sem_i
        ).wait()

        def start_gather(chunk, slot):
            pltpu.async_copy(t_hbm.at[iv.at[chunk]], gb.at[slot], sem_g[slot])

        def wait_gather(slot):
            pltpu.make_async_copy(
                t_hbm.at[iv.at[0]], gb.at[slot], sem_g[slot]).wait()

        def scale(slot):
            @pl.loop(0, CHUNK)
            def _r(r):
                for k in range(EMBED // LANES):
                    sl = pl.ds(k * LANES, LANES)
                    ob.at[slot, r, sl][...] = gb.at[slot, r, sl][...] * SCALE

        def start_out(chunk, slot):
            pltpu.async_copy(
                ob.at[slot],
                o_hbm.at[pl.ds(base + chunk * CHUNK, CHUNK), pl.ds(0, EMBED)],
                sem_o[slot],
            )

        def wait_out(chunk, slot):
            pltpu.make_async_copy(
                ob.at[slot],
                o_hbm.at[pl.ds(base + chunk * CHUNK, CHUNK), pl.ds(0, EMBED)],
                sem_o[slot],
            ).wait()

        start_gather(0, 0)

        @pl.loop(0, n_chunks // 2)
        def _t(t):
            for b in range(2):
                i = t * 2 + b
                nxt = 1 - b

                @pl.when(i + 1 < n_chunks)
                def _():
                    start_gather(i + 1, nxt)

                wait_gather(b)

                @pl.when(i >= 2)
                def _():
                    wait_out(i - 2, b)

                scale(b)
                start_out(i, b)

        for b in range(2):
            wait_out(n_chunks - 2 + b, b)

    out = gather_scale(idx, table)
    return out[:, :EMBED].reshape(B, L, EMBED)


# speedup vs baseline: 2.4182x; 1.0281x over previous
import functools

import jax
import jax.numpy as jnp
from jax import lax
from jax.experimental import pallas as pl
from jax.experimental.pallas import tpu as pltpu
from jax.experimental.pallas import tpu_sc as plsc

EMBED = 64
SCALE = 8.0
LANES = 16
CHUNK = 256
NWORK = 32


def kernel(x, table):
    B, L = x.shape
    N = B * L
    per_w = N // NWORK
    n_chunks = per_w // CHUNK
    rows_w = per_w // CHUNK

    idx = x.reshape(N // CHUNK, CHUNK)

    mesh = plsc.VectorSubcoreMesh(core_axis_name="c", subcore_axis_name="s")

    @functools.partial(
        pl.kernel,
        out_type=jax.ShapeDtypeStruct((N, 2 * EMBED), jnp.float32),
        mesh=mesh,
        compiler_params=pltpu.CompilerParams(use_tc_tiling_on_sc=False),
        scratch_types=[
            pltpu.VMEM((rows_w, CHUNK), jnp.int32),
            pltpu.VMEM((2, CHUNK, EMBED), jnp.float32),
            pltpu.VMEM((2, CHUNK, EMBED), jnp.float32),
            pltpu.SemaphoreType.DMA,
            pltpu.SemaphoreType.DMA,
            pltpu.SemaphoreType.DMA,
            pltpu.SemaphoreType.DMA,
            pltpu.SemaphoreType.DMA,
        ],
    )
    def gather_scale(i_hbm, t_hbm, o_hbm, iv, gb, ob, sem_i, sg0, sg1, so0,
                     so1):
        wid = lax.axis_index("s") * 2 + lax.axis_index("c")
        base = wid * per_w
        sem_g = (sg0, sg1)
        sem_o = (so0, so1)

        pltpu.async_copy(
            i_hbm.at[pl.ds(wid * rows_w, rows_w), :], iv, sem_i
        ).wait()

        def start_gather(chunk, slot):
            pltpu.async_copy(t_hbm.at[iv.at[chunk]], gb.at[slot], sem_g[slot])

        def wait_gather(slot):
            pltpu.make_async_copy(
                t_hbm.at[iv.at[0]], gb.at[slot], sem_g[slot]).wait()

        def scale(slot):
            @pl.loop(0, CHUNK)
            def _r(r):
                for k in range(EMBED // LANES):
                    sl = pl.ds(k * LANES, LANES)
                    ob.at[slot, r, sl][...] = gb.at[slot, r, sl][...] * SCALE

        def start_out(chunk, slot):
            pltpu.async_copy(
                ob.at[slot],
                o_hbm.at[pl.ds(base + chunk * CHUNK, CHUNK), pl.ds(0, EMBED)],
                sem_o[slot],
            )

        def wait_out(chunk, slot):
            pltpu.make_async_copy(
                ob.at[slot],
                o_hbm.at[pl.ds(base + chunk * CHUNK, CHUNK), pl.ds(0, EMBED)],
                sem_o[slot],
            ).wait()

        start_gather(0, 0)

        @pl.loop(0, n_chunks // 2)
        def _t(t):
            for b in range(2):
                i = t * 2 + b
                nxt = 1 - b

                @pl.when(i + 1 < n_chunks)
                def _():
                    start_gather(i + 1, nxt)

                wait_gather(b)

                @pl.when(i >= 2)
                def _():
                    wait_out(i - 2, b)

                scale(b)
                start_out(i, b)

        for b in range(2):
            wait_out(n_chunks - 2 + b, b)

    out = gather_scale(idx, table)
    return out[:, :EMBED].reshape(B, L, EMBED)
